# baseline (speedup 1.0000x reference)
import jax
import jax.numpy as jnp
from jax import lax
from jax.experimental import pallas as pl
from jax.experimental.pallas import tpu as pltpu
from jax.experimental.pallas import tpu_sc as plsc

N, C, H, W = 8, 128, 224, 224
PH, PW = H + 2, W + 2
P = H * W
TOTAL = N * P
CW = C // 2
PPR = PW // 2
PAIRS_PER_IMG = PH * PPR
NPAIR = N * PAIRS_PER_IMG
NROWS = NPAIR + 8
NC, NS = 2, 16
NW = NC * NS
PER_TILE = TOTAL // NW
CHUNK = 64
NCHUNK = PER_TILE // CHUNK
PAIRS = NCHUNK // 2
SUBS = CHUNK // 16
L = 16


def _sc_body(table, xs_hbm, ys_hbm, out_hbm,
             xs_v, ys_v,
             i00, i10, u00, u10, u20, v00, v10, v20, b00, b10,
             i01, i11, u01, u11, u21, v01, v11, v21, b01, b11,
             ob,
             s00, s10, s01, s11, so):
    setA = (i00, i10, u00, u10, u20, v00, v10, v20, b00, b10, s00, s10)
    setB = (i01, i11, u01, u11, u21, v01, v11, v21, b01, b11, s01, s11)

    wid = lax.axis_index("s") * NC + lax.axis_index("c")
    base_g = wid * PER_TILE
    row_base = (wid // 4) * PAIRS_PER_IMG

    pltpu.sync_copy(xs_hbm.at[pl.ds(base_g, PER_TILE)], xs_v)
    pltpu.sync_copy(ys_hbm.at[pl.ds(base_g, PER_TILE)], ys_v)

    iota = lax.iota(jnp.int32, L)

    def fire(chk, S):
        i0, i1, u0, u1, u2, v0, v1, v2, b0, b1, s0, s1 = S
        off = chk * CHUNK
        for s in range(SUBS):
            xv = xs_v[pl.ds(off + s * L, L)]
            yv = ys_v[pl.ds(off + s * L, L)]
            x = ((xv + 1.0) * W - 1.0) * 0.5
            y = ((yv + 1.0) * H - 1.0) * 0.5
            xi = x.astype(jnp.int32)
            yi = y.astype(jnp.int32)
            x0 = jnp.where(xi.astype(jnp.float32) > x, xi - 1, xi)
            y0 = jnp.where(yi.astype(jnp.float32) > y, yi - 1, yi)
            x0f = x0.astype(jnp.float32)
            y0f = y0.astype(jnp.float32)
            dx1 = (x0f + 1.0) - x
            dx0 = x - x0f
            dy1 = (y0f + 1.0) - y
            dy0 = y - y0f
            xp = x0 + 1
            flat = (y0 + 1) * PW + xp
            q = row_base + lax.shift_right_logical(flat, 1)
            po = (xp & 1).astype(jnp.float32)
            pe = 1.0 - po
            wa = dx1 * dy1
            wc = dx0 * dy1
            wb = dx1 * dy0
            wd = dx0 * dy0
            pos = (iota + s * L) * 2
            plsc.store_scatter(i0, [pos], q)
            plsc.store_scatter(i0, [pos + 1], q + 1)
            plsc.store_scatter(i1, [pos], q + PPR)
            plsc.store_scatter(i1, [pos + 1], q + PPR + 1)
            u0[pl.ds(s * L, L)] = wa * pe
            u1[pl.ds(s * L, L)] = wa * po + wc * pe
            u2[pl.ds(s * L, L)] = wc * po
            v0[pl.ds(s * L, L)] = wb * pe
            v1[pl.ds(s * L, L)] = wb * po + wd * pe
            v2[pl.ds(s * L, L)] = wd * po
        pltpu.async_copy(table.at[i0], b0, s0)
        pltpu.async_copy(table.at[i1], b1, s1)

    def wait_gathers(S):
        i0, i1, _, _, _, _, _, _, b0, b1, s0, s1 = S
        pltpu.make_async_copy(table.at[i0], b0, s0).wait()
        pltpu.make_async_copy(table.at[i1], b1, s1).wait()

    def combine(S, half):
        _, _, u0, u1, u2, v0, v1, v2, b0, b1, _, _ = S
        fmt = plsc.PackFormat.INTERLEAVED

        def p_body(p, c2):
            pv = jnp.full((L,), 0, jnp.int32) + p
            su0 = plsc.load_gather(u0, [pv])
            su1 = plsc.load_gather(u1, [pv])
            su2 = plsc.load_gather(u2, [pv])
            sv0 = plsc.load_gather(v0, [pv])
            sv1 = plsc.load_gather(v1, [pv])
            sv2 = plsc.load_gather(v2, [pv])
            pu0 = plsc.pack(su0, su0, format=fmt)
            pu1 = plsc.pack(su1, su1, format=fmt)
            pu2 = plsc.pack(su2, su2, format=fmt)
            pv0 = plsc.pack(sv0, sv0, format=fmt)
            pv1 = plsc.pack(sv1, sv1, format=fmt)
            pv2 = plsc.pack(sv2, sv2, format=fmt)
            r0 = 2 * p
            q = p + half * CHUNK
            for w4 in range(CW // L):
                sl0 = pl.ds(w4 * L, L)
                sl1 = pl.ds(CW + w4 * L, L)
                t0 = plsc.bitcast(b0[r0, sl0], jnp.bfloat16)
                t1 = plsc.bitcast(b0[r0, sl1], jnp.bfloat16)
                t2 = plsc.bitcast(b0[r0 + 1, sl0], jnp.bfloat16)
                g0 = plsc.bitcast(b1[r0, sl0], jnp.bfloat16)
                g1 = plsc.bitcast(b1[r0, sl1], jnp.bfloat16)
                g2 = plsc.bitcast(b1[r0 + 1, sl0], jnp.bfloat16)
                acc = (((t0 * pu0 + t1 * pu1) + (t2 * pu2 + g0 * pv0))
                       + (g1 * pv1 + g2 * pv2))
                ob[q, sl0] = plsc.bitcast(acc, jnp.int32)
            return c2

        lax.fori_loop(0, CHUNK, p_body, 0, unroll=2)

    def out_copy(k):
        return pltpu.make_async_copy(
            ob, out_hbm.at[pl.ds(base_g + k * (2 * CHUNK), 2 * CHUNK)], so)

    fire(0, setA)
    out_copy(0).start()

    def step(k, carry):
        c0 = 2 * k
        fire(c0 + 1, setB)
        wait_gathers(setA)
        out_copy(k).wait()
        combine(setA, 0)
        nxt = jnp.minimum(c0 + 2, NCHUNK - 2)
        fire(nxt, setA)
        wait_gathers(setB)
        combine(setB, 1)
        out_copy(k).start()
        return carry

    lax.fori_loop(0, PAIRS, step, 0)

    out_copy(0).wait()
    wait_gathers(setA)


def _scratch_set():
    return (
        [pltpu.VMEM((2 * CHUNK,), jnp.int32) for _ in range(2)]
        + [pltpu.VMEM((CHUNK,), jnp.float32) for _ in range(6)]
        + [pltpu.VMEM((2 * CHUNK, 2 * CW), jnp.int32) for _ in range(2)]
    )


_sc_sample = pl.kernel(
    _sc_body,
    out_type=jax.ShapeDtypeStruct((TOTAL, CW), jnp.int32),
    mesh=plsc.VectorSubcoreMesh(
        core_axis_name="c", subcore_axis_name="s",
        num_cores=NC, num_subcores=NS),
    compiler_params=pltpu.CompilerParams(needs_layout_passes=False),
    scratch_types=(
        [pltpu.VMEM((PER_TILE,), jnp.float32),
         pltpu.VMEM((PER_TILE,), jnp.float32)]
        + _scratch_set()
        + _scratch_set()
        + [pltpu.VMEM((2 * CHUNK, CW), jnp.int32)]
        + [pltpu.SemaphoreType.DMA] * 5
    ),
)


def _to_bf16_bits(x):
    u = jax.lax.bitcast_convert_type(x, jnp.int32)
    lsb = jax.lax.shift_right_logical(u, 16) & 1
    return jax.lax.shift_right_logical(u + 0x7FFF + lsb, 16)


@jax.jit
def kernel(img, points):
    n, c, h, w = img.shape
    tl = jnp.pad(
        img.transpose(0, 2, 3, 1), ((0, 0), (1, 1), (1, 1), (0, 0)))
    tr = tl.reshape(n, PH, PPR, 2 * c)
    w0 = _to_bf16_bits(tr[..., 0:CW]) | jnp.left_shift(
        _to_bf16_bits(tr[..., CW:c]), 16)
    w1 = _to_bf16_bits(tr[..., c:c + CW]) | jnp.left_shift(
        _to_bf16_bits(tr[..., c + CW:]), 16)
    table = jnp.pad(
        jnp.concatenate([w0, w1], axis=-1).reshape(NPAIR, 2 * CW),
        ((0, NROWS - NPAIR), (0, 0)))
    xs = points[..., 0].reshape(-1)
    ys = points[..., 1].reshape(-1)
    out_w = _sc_sample(table, xs, ys).reshape(n, h, w, CW)
    lo = jax.lax.bitcast_convert_type(
        jnp.left_shift(out_w, 16), jnp.float32)
    hi = jax.lax.bitcast_convert_type(
        out_w & jnp.int32(-65536), jnp.float32)
    out_t = jnp.concatenate([lo, hi], axis=-1)
    return out_t.transpose(0, 3, 1, 2)


# speedup vs baseline: 3.2280x; 3.2280x over previous
import jax
import jax.numpy as jnp
from jax import lax
from jax.experimental import pallas as pl
from jax.experimental.pallas import tpu as pltpu
from jax.experimental.pallas import tpu_sc as plsc

N, C, H, W = 8, 128, 224, 224
PH, PW = H + 2, W + 2
ROWS_PER_IMG = PH * PW
P = H * W
TOTAL = N * P
NC, NS = 2, 16
NW = NC * NS
PER_TILE = TOTAL // NW
CHUNK = 64
NCHUNK = PER_TILE // CHUNK
PAIRS = NCHUNK // 2
SUBS = CHUNK // 16
L = 16


def _sc_body(table, xs_hbm, ys_hbm, out_hbm,
             xs_v, ys_v,
             ia0, ib0, ic0, id0, wa0, wb0, wc0, wd0, ba0, bb0, bc0, bd0,
             ia1, ib1, ic1, id1, wa1, wb1, wc1, wd1, ba1, bb1, bc1, bd1,
             ob,
             sa0, sb0, sc0, sd0, sa1, sb1, sc1, sd1, so):
    setA = (ia0, ib0, ic0, id0, wa0, wb0, wc0, wd0, ba0, bb0, bc0, bd0,
            sa0, sb0, sc0, sd0)
    setB = (ia1, ib1, ic1, id1, wa1, wb1, wc1, wd1, ba1, bb1, bc1, bd1,
            sa1, sb1, sc1, sd1)

    wid = lax.axis_index("s") * NC + lax.axis_index("c")
    base_g = wid * PER_TILE
    row_base = (wid // 4) * ROWS_PER_IMG

    pltpu.sync_copy(xs_hbm.at[pl.ds(base_g, PER_TILE)], xs_v)
    pltpu.sync_copy(ys_hbm.at[pl.ds(base_g, PER_TILE)], ys_v)

    def fire(chk, S):
        ia, ib, ic, id_, wa_v, wb_v, wc_v, wd_v, ba, bb, bc, bd, \
            sa, sb, sc, sd = S
        off = chk * CHUNK
        for s in range(SUBS):
            xv = xs_v[pl.ds(off + s * L, L)]
            yv = ys_v[pl.ds(off + s * L, L)]
            x = ((xv + 1.0) * W - 1.0) * 0.5
            y = ((yv + 1.0) * H - 1.0) * 0.5
            xi = x.astype(jnp.int32)
            yi = y.astype(jnp.int32)
            x0 = jnp.where(xi.astype(jnp.float32) > x, xi - 1, xi)
            y0 = jnp.where(yi.astype(jnp.float32) > y, yi - 1, yi)
            x0f = x0.astype(jnp.float32)
            y0f = y0.astype(jnp.float32)
            dx1 = (x0f + 1.0) - x
            dx0 = x - x0f
            dy1 = (y0f + 1.0) - y
            dy0 = y - y0f
            base = row_base + (y0 + 1) * PW + (x0 + 1)
            ia[pl.ds(s * L, L)] = base
            ic[pl.ds(s * L, L)] = base + 1
            ib[pl.ds(s * L, L)] = base + PW
            id_[pl.ds(s * L, L)] = base + PW + 1
            wa_v[pl.ds(s * L, L)] = dx1 * dy1
            wb_v[pl.ds(s * L, L)] = dx1 * dy0
            wc_v[pl.ds(s * L, L)] = dx0 * dy1
            wd_v[pl.ds(s * L, L)] = dx0 * dy0
        pltpu.async_copy(table.at[ia], ba, sa)
        pltpu.async_copy(table.at[ib], bb, sb)
        pltpu.async_copy(table.at[ic], bc, sc)
        pltpu.async_copy(table.at[id_], bd, sd)

    def wait_gathers(S):
        ia, ib, ic, id_, _, _, _, _, ba, bb, bc, bd, sa, sb, sc, sd = S
        pltpu.make_async_copy(table.at[ia], ba, sa).wait()
        pltpu.make_async_copy(table.at[ib], bb, sb).wait()
        pltpu.make_async_copy(table.at[ic], bc, sc).wait()
        pltpu.make_async_copy(table.at[id_], bd, sd).wait()

    def combine(S, half):
        _, _, _, _, wa_v, wb_v, wc_v, wd_v, ba, bb, bc, bd, \
            _, _, _, _ = S

        @plsc.parallel_loop(0, CHUNK, unroll=2)
        def p_body(p):
            pv = jnp.full((L,), 0, jnp.int32) + p
            wav = plsc.load_gather(wa_v, [pv])
            wbv = plsc.load_gather(wb_v, [pv])
            wcv = plsc.load_gather(wc_v, [pv])
            wdv = plsc.load_gather(wd_v, [pv])
            q = p + (half * CHUNK)
            for c8 in range(C // L):
                sl = pl.ds(c8 * L, L)
                ob[q, sl] = ((ba[p, sl] * wav + bb[p, sl] * wbv)
                             + (bc[p, sl] * wcv + bd[p, sl] * wdv))

    def out_copy(k):
        return pltpu.make_async_copy(
            ob, out_hbm.at[pl.ds(base_g + k * (2 * CHUNK), 2 * CHUNK)], so)

    fire(0, setA)
    out_copy(0).start()

    def step(k, carry):
        c0 = 2 * k
        fire(c0 + 1, setB)
        wait_gathers(setA)
        out_copy(k).wait()
        combine(setA, 0)
        nxt = jnp.minimum(c0 + 2, NCHUNK - 2)
        fire(nxt, setA)
        wait_gathers(setB)
        combine(setB, 1)
        out_copy(k).start()
        return carry

    lax.fori_loop(0, PAIRS, step, 0)

    out_copy(0).wait()
    wait_gathers(setA)


def _scratch_set():
    return (
        [pltpu.VMEM((CHUNK,), jnp.int32) for _ in range(4)]
        + [pltpu.VMEM((CHUNK,), jnp.float32) for _ in range(4)]
        + [pltpu.VMEM((CHUNK, C), jnp.float32) for _ in range(4)]
    )


_sc_sample = pl.kernel(
    _sc_body,
    out_type=jax.ShapeDtypeStruct((TOTAL, C), jnp.float32),
    mesh=plsc.VectorSubcoreMesh(
        core_axis_name="c", subcore_axis_name="s",
        num_cores=NC, num_subcores=NS),
    compiler_params=pltpu.CompilerParams(needs_layout_passes=False),
    scratch_types=(
        [pltpu.VMEM((PER_TILE,), jnp.float32),
         pltpu.VMEM((PER_TILE,), jnp.float32)]
        + _scratch_set()
        + _scratch_set()
        + [pltpu.VMEM((2 * CHUNK, C), jnp.float32)]
        + [pltpu.SemaphoreType.DMA] * 9
    ),
)


@jax.jit
def kernel(img, points):
    n, c, h, w = img.shape
    table = jnp.pad(
        img.transpose(0, 2, 3, 1), ((0, 0), (1, 1), (1, 1), (0, 0))
    ).reshape(n * (h + 2) * (w + 2), c)
    xs = points[..., 0].reshape(-1)
    ys = points[..., 1].reshape(-1)
    out_t = _sc_sample(table, xs, ys)
    return out_t.reshape(n, h, w, c).transpose(0, 3, 1, 2)


# speedup vs baseline: 5.5609x; 1.7227x over previous
import jax
import jax.numpy as jnp
from jax import lax
from jax.experimental import pallas as pl
from jax.experimental.pallas import tpu as pltpu
from jax.experimental.pallas import tpu_sc as plsc

N, C, H, W = 8, 128, 224, 224
PH, PW = H + 2, W + 2
ROWS_PER_IMG = PH * PW
P = H * W
TOTAL = N * P
NC, NS = 2, 16
NW = NC * NS
PER_TILE = TOTAL // NW
CHUNK = 64
NCHUNK = PER_TILE // CHUNK
PAIRS = NCHUNK // 2
SUBS = CHUNK // 16
L = 16


def _sc_body(table, xs_hbm, ys_hbm, out_hbm,
             xs_v, ys_v,
             ia0, ib0, ic0, id0, wa0, wb0, wc0, wd0, ba0, bb0, bc0, bd0,
             ia1, ib1, ic1, id1, wa1, wb1, wc1, wd1, ba1, bb1, bc1, bd1,
             ob,
             sa0, sb0, sc0, sd0, sa1, sb1, sc1, sd1, so):
    setA = (ia0, ib0, ic0, id0, wa0, wb0, wc0, wd0, ba0, bb0, bc0, bd0,
            sa0, sb0, sc0, sd0)
    setB = (ia1, ib1, ic1, id1, wa1, wb1, wc1, wd1, ba1, bb1, bc1, bd1,
            sa1, sb1, sc1, sd1)

    wid = lax.axis_index("s") * NC + lax.axis_index("c")
    base_g = wid * PER_TILE
    row_base = (wid // 4) * P

    pltpu.sync_copy(xs_hbm.at[pl.ds(base_g, PER_TILE)], xs_v)
    pltpu.sync_copy(ys_hbm.at[pl.ds(base_g, PER_TILE)], ys_v)

    def fire(chk, S):
        ia, ib, ic, id_, wa_v, wb_v, wc_v, wd_v, ba, bb, bc, bd, \
            sa, sb, sc, sd = S
        off = chk * CHUNK
        for s in range(SUBS):
            xv = xs_v[pl.ds(off + s * L, L)]
            yv = ys_v[pl.ds(off + s * L, L)]
            x = ((xv + 1.0) * W - 1.0) * 0.5
            y = ((yv + 1.0) * H - 1.0) * 0.5
            xi = x.astype(jnp.int32)
            yi = y.astype(jnp.int32)
            x0 = jnp.where(xi.astype(jnp.float32) > x, xi - 1, xi)
            y0 = jnp.where(yi.astype(jnp.float32) > y, yi - 1, yi)
            x0f = x0.astype(jnp.float32)
            y0f = y0.astype(jnp.float32)
            dx1 = (x0f + 1.0) - x
            dx0 = x - x0f
            dy1 = (y0f + 1.0) - y
            dy0 = y - y0f
            one = jnp.float32(1.0)
            zero = jnp.float32(0.0)
            vx0 = jnp.where(x0 >= 0, one, zero)
            vy0 = jnp.where(y0 >= 0, one, zero)
            vx1 = jnp.where(x0 < W - 1, one, zero)
            vy1 = jnp.where(y0 < H - 1, one, zero)
            x0c = jnp.maximum(x0, 0)
            y0c = jnp.maximum(y0, 0)
            x1c = jnp.minimum(x0 + 1, W - 1)
            y1c = jnp.minimum(y0 + 1, H - 1)
            r0 = row_base + y0c * W
            r1 = row_base + y1c * W
            ia[pl.ds(s * L, L)] = r0 + x0c
            ic[pl.ds(s * L, L)] = r0 + x1c
            ib[pl.ds(s * L, L)] = r1 + x0c
            id_[pl.ds(s * L, L)] = r1 + x1c
            wa_v[pl.ds(s * L, L)] = dx1 * dy1 * (vx0 * vy0)
            wb_v[pl.ds(s * L, L)] = dx1 * dy0 * (vx0 * vy1)
            wc_v[pl.ds(s * L, L)] = dx0 * dy1 * (vx1 * vy0)
            wd_v[pl.ds(s * L, L)] = dx0 * dy0 * (vx1 * vy1)
        pltpu.async_copy(table.at[ia], ba, sa)
        pltpu.async_copy(table.at[ib], bb, sb)
        pltpu.async_copy(table.at[ic], bc, sc)
        pltpu.async_copy(table.at[id_], bd, sd)

    def wait_gathers(S):
        ia, ib, ic, id_, _, _, _, _, ba, bb, bc, bd, sa, sb, sc, sd = S
        pltpu.make_async_copy(table.at[ia], ba, sa).wait()
        pltpu.make_async_copy(table.at[ib], bb, sb).wait()
        pltpu.make_async_copy(table.at[ic], bc, sc).wait()
        pltpu.make_async_copy(table.at[id_], bd, sd).wait()

    def combine(S, half):
        _, _, _, _, wa_v, wb_v, wc_v, wd_v, ba, bb, bc, bd, \
            _, _, _, _ = S

        @plsc.parallel_loop(0, CHUNK, unroll=2)
        def p_body(p):
            pv = jnp.full((L,), 0, jnp.int32) + p
            wav = plsc.load_gather(wa_v, [pv])
            wbv = plsc.load_gather(wb_v, [pv])
            wcv = plsc.load_gather(wc_v, [pv])
            wdv = plsc.load_gather(wd_v, [pv])
            q = p + (half * CHUNK)
            for c8 in range(C // L):
                sl = pl.ds(c8 * L, L)
                ob[q, sl] = ((ba[p, sl] * wav + bb[p, sl] * wbv)
                             + (bc[p, sl] * wcv + bd[p, sl] * wdv))

    def out_copy(k):
        return pltpu.make_async_copy(
            ob, out_hbm.at[pl.ds(base_g + k * (2 * CHUNK), 2 * CHUNK)], so)

    fire(0, setA)
    out_copy(0).start()

    def step(k, carry):
        c0 = 2 * k
        fire(c0 + 1, setB)
        wait_gathers(setA)
        out_copy(k).wait()
        combine(setA, 0)
        nxt = jnp.minimum(c0 + 2, NCHUNK - 2)
        fire(nxt, setA)
        wait_gathers(setB)
        combine(setB, 1)
        out_copy(k).start()
        return carry

    lax.fori_loop(0, PAIRS, step, 0)

    out_copy(0).wait()
    wait_gathers(setA)


def _scratch_set():
    return (
        [pltpu.VMEM((CHUNK,), jnp.int32) for _ in range(4)]
        + [pltpu.VMEM((CHUNK,), jnp.float32) for _ in range(4)]
        + [pltpu.VMEM((CHUNK, C), jnp.float32) for _ in range(4)]
    )


_sc_sample = pl.kernel(
    _sc_body,
    out_type=jax.ShapeDtypeStruct((TOTAL, C), jnp.float32),
    mesh=plsc.VectorSubcoreMesh(
        core_axis_name="c", subcore_axis_name="s",
        num_cores=NC, num_subcores=NS),
    compiler_params=pltpu.CompilerParams(needs_layout_passes=False),
    scratch_types=(
        [pltpu.VMEM((PER_TILE,), jnp.float32),
         pltpu.VMEM((PER_TILE,), jnp.float32)]
        + _scratch_set()
        + _scratch_set()
        + [pltpu.VMEM((2 * CHUNK, C), jnp.float32)]
        + [pltpu.SemaphoreType.DMA] * 9
    ),
)


@jax.jit
def kernel(img, points):
    n, c, h, w = img.shape
    table = img.transpose(0, 2, 3, 1).reshape(n * h * w, c)
    xs = points[..., 0].reshape(-1)
    ys = points[..., 1].reshape(-1)
    out_t = _sc_sample(table, xs, ys)
    return out_t.reshape(n, h, w, c).transpose(0, 3, 1, 2)
